# baseline (speedup 1.0000x reference)
import functools

import jax
import jax.numpy as jnp
import numpy as np
from jax import lax
from jax.experimental import pallas as pl
from jax.experimental.pallas import tpu as pltpu
from jax.experimental.pallas import tpu_sc as plsc

_PATCH = 32
_GRID = 10
_H2 = 16
_W2 = 16
_E = _PATCH * _PATCH * 3
_WHALF = 256


def _hash_rows():
    i = np.arange(_H2)
    j = np.arange(_W2)
    hi = np.floor(i.astype(np.float32) * _GRID / _H2).astype(np.int32)
    hj = np.floor(j.astype(np.float32) * _GRID / _W2).astype(np.int32)
    return (hi[:, None] * _GRID + hj[None, :]).reshape(-1)


def _sc_body(x_hbm, tab_hbm, idx_hbm, out_hbm,
             in0, in1, out0, out1, shared, idx8_v,
             sem_i0, sem_i1, sem_o0, sem_o1, sem_p0, sem_p1, sem_t):
    cidx = lax.axis_index("c")
    sidx = lax.axis_index("s")
    wid = sidx * 2 + cidx
    ph = wid // 2
    half = wid % 2
    p0 = ph * _W2 + half * 8
    row0 = ph * _PATCH
    w0 = half * _WHALF
    nb = x_hbm.shape[0]

    def din(b, buf, sem):
        return pltpu.make_async_copy(
            x_hbm.at[b, :, pl.ds(row0, _PATCH), pl.ds(w0, _WHALF)], buf, sem)

    def dout(b, buf, sem):
        return pltpu.make_async_copy(buf, out_hbm.at[b, pl.ds(p0, 8)], sem)

    def prefill(buf, sem):
        return pltpu.make_async_copy(shared.at[sidx], buf, sem)

    din(0, in0, sem_i0).start()
    din(1, in1, sem_i1).start()

    pltpu.sync_copy(idx_hbm.at[pl.ds(p0, 8)], idx8_v)
    pltpu.async_copy(tab_hbm.at[idx8_v], out0, sem_t).wait()
    pltpu.sync_copy(out0, shared.at[sidx])

    prefill(out0, sem_p0).start()
    prefill(out1, sem_p1).start()

    iota3 = lax.iota(jnp.int32, 16) * 3
    rows = [jnp.full((16,), pw, jnp.int32) for pw in range(8)]

    def compute_row(in_v, out_v, pw):
        row = rows[pw]

        @plsc.parallel_loop(0, _PATCH, unroll=1)
        def _(kh):
            base = kh * 96
            for c in range(3):
                for sh in range(2):
                    col = iota3 + (base + c + sh * 48)
                    v = in_v[c, kh, pl.ds(pw * 32 + sh * 16, 16)]
                    plsc.addupdate_scatter(out_v, [row, col], v)

    def dout_row(b, buf, pw, sem):
        return pltpu.make_async_copy(buf.at[pw], out_hbm.at[b, p0 + pw], sem)

    def phase(b, in_v, out_v, sem_i, sem_o, sem_p):
        din(0, in_v, sem_i).wait()
        prefill(out_v, sem_p).wait()
        for pw in range(8):
            compute_row(in_v, out_v, pw)
            dout_row(b, out_v, pw, sem_o).start()

        @pl.when(b + 2 < nb)
        def _():
            din(b + 2, in_v, sem_i).start()

    phase(0, in0, out0, sem_i0, sem_o0, sem_p0)
    phase(1, in1, out1, sem_i1, sem_o1, sem_p1)

    def steady(i, carry):
        dout(0, out0, sem_o0).wait()
        prefill(out0, sem_p0).start()
        dout(0, out1, sem_o1).wait()
        prefill(out1, sem_p1).start()
        phase(2 * i, in0, out0, sem_i0, sem_o0, sem_p0)
        phase(2 * i + 1, in1, out1, sem_i1, sem_o1, sem_p1)
        return carry

    lax.fori_loop(1, nb // 2, steady, 0)

    dout(0, out0, sem_o0).wait()
    dout(0, out1, sem_o1).wait()


def kernel(x, position_embeddings):
    b = x.shape[0]
    table = position_embeddings.reshape(_GRID * _GRID, _E)
    idx = jnp.asarray(_hash_rows())

    mesh = plsc.VectorSubcoreMesh(core_axis_name="c", subcore_axis_name="s")
    run = functools.partial(
        pl.kernel,
        out_type=jax.ShapeDtypeStruct((b, _H2 * _W2, _E), x.dtype),
        mesh=mesh,
        compiler_params=pltpu.CompilerParams(needs_layout_passes=False),
        scratch_types=[
            pltpu.VMEM((3, _PATCH, _WHALF), jnp.float32),
            pltpu.VMEM((3, _PATCH, _WHALF), jnp.float32),
            pltpu.VMEM((8, _E), jnp.float32),
            pltpu.VMEM((8, _E), jnp.float32),
            pltpu.VMEM_SHARED((16, 8, _E), jnp.float32),
            pltpu.VMEM((8,), jnp.int32),
            pltpu.SemaphoreType.DMA,
            pltpu.SemaphoreType.DMA,
            pltpu.SemaphoreType.DMA,
            pltpu.SemaphoreType.DMA,
            pltpu.SemaphoreType.DMA,
            pltpu.SemaphoreType.DMA,
            pltpu.SemaphoreType.DMA,
        ],
    )(_sc_body)
    return run(x, table, idx)


# speedup vs baseline: 1.1565x; 1.1565x over previous
import functools

import jax
import jax.numpy as jnp
import numpy as np
from jax import lax
from jax.experimental import pallas as pl
from jax.experimental.pallas import tpu as pltpu
from jax.experimental.pallas import tpu_sc as plsc

_PATCH = 32
_GRID = 10
_H2 = 16
_W2 = 16
_E = _PATCH * _PATCH * 3
_WHALF = 256


def _hash_rows():
    i = np.arange(_H2)
    j = np.arange(_W2)
    hi = np.floor(i.astype(np.float32) * _GRID / _H2).astype(np.int32)
    hj = np.floor(j.astype(np.float32) * _GRID / _W2).astype(np.int32)
    return (hi[:, None] * _GRID + hj[None, :]).reshape(-1)


def _sc_body(x_hbm, tab_hbm, idx_hbm, out_hbm,
             in0, in1, out0, out1, shared, idx8_v,
             sem_i0, sem_i1, sem_o0, sem_o1, sem_p0, sem_p1, sem_t):
    cidx = lax.axis_index("c")
    sidx = lax.axis_index("s")
    wid = sidx * 2 + cidx
    ph = wid // 2
    half = wid % 2
    p0 = ph * _W2 + half * 8
    row0 = ph * _PATCH
    w0 = half * _WHALF
    nb = x_hbm.shape[0]

    def din(b, buf, sem):
        return pltpu.make_async_copy(
            x_hbm.at[b, :, pl.ds(row0, _PATCH), pl.ds(w0, _WHALF)], buf, sem)

    def dout(b, buf, sem):
        return pltpu.make_async_copy(buf, out_hbm.at[b, pl.ds(p0, 8)], sem)

    def prefill(buf, sem):
        return pltpu.make_async_copy(shared.at[sidx], buf, sem)

    din(0, in0, sem_i0).start()
    din(1, in1, sem_i1).start()

    pltpu.sync_copy(idx_hbm.at[pl.ds(p0, 8)], idx8_v)
    pltpu.async_copy(tab_hbm.at[idx8_v], out0, sem_t).wait()
    pltpu.sync_copy(out0, shared.at[sidx])

    prefill(out0, sem_p0).start()
    prefill(out1, sem_p1).start()

    iota3 = lax.iota(jnp.int32, 16) * 3
    rows = [jnp.full((16,), pw, jnp.int32) for pw in range(8)]

    def compute(in_v, out_v, lo, hi):
        for c in range(3):
            @plsc.parallel_loop(0, _PATCH, unroll=1)
            def _(kh, c=c):
                col0 = iota3 + (kh * 96 + c)
                col1 = col0 + 48
                for s in range(2 * lo, 2 * hi):
                    row = rows[s // 2]
                    col = col0 if s % 2 == 0 else col1
                    v = in_v[c, kh, pl.ds(s * 16, 16)]
                    plsc.addupdate_scatter(out_v, [row, col], v)

    def dhalf(b, buf, r, sem):
        return pltpu.make_async_copy(
            buf.at[pl.ds(r, 4)], out_hbm.at[b, pl.ds(p0 + r, 4)], sem)

    def phase(b, in_v, out_v, sem_i, sem_o, sem_p):
        din(0, in_v, sem_i).wait()
        prefill(out_v, sem_p).wait()
        compute(in_v, out_v, 0, 4)
        dhalf(b, out_v, 0, sem_o).start()
        compute(in_v, out_v, 4, 8)
        dhalf(b, out_v, 4, sem_o).start()

        @pl.when(b + 2 < nb)
        def _():
            din(b + 2, in_v, sem_i).start()

    phase(0, in0, out0, sem_i0, sem_o0, sem_p0)
    phase(1, in1, out1, sem_i1, sem_o1, sem_p1)

    def steady(i, carry):
        dout(0, out0, sem_o0).wait()
        prefill(out0, sem_p0).start()
        dout(0, out1, sem_o1).wait()
        prefill(out1, sem_p1).start()
        phase(2 * i, in0, out0, sem_i0, sem_o0, sem_p0)
        phase(2 * i + 1, in1, out1, sem_i1, sem_o1, sem_p1)
        return carry

    lax.fori_loop(1, nb // 2, steady, 0)

    dout(0, out0, sem_o0).wait()
    dout(0, out1, sem_o1).wait()


def kernel(x, position_embeddings):
    b = x.shape[0]
    table = position_embeddings.reshape(_GRID * _GRID, _E)
    idx = jnp.asarray(_hash_rows())

    mesh = plsc.VectorSubcoreMesh(core_axis_name="c", subcore_axis_name="s")
    run = functools.partial(
        pl.kernel,
        out_type=jax.ShapeDtypeStruct((b, _H2 * _W2, _E), x.dtype),
        mesh=mesh,
        compiler_params=pltpu.CompilerParams(needs_layout_passes=False),
        scratch_types=[
            pltpu.VMEM((3, _PATCH, _WHALF), jnp.float32),
            pltpu.VMEM((3, _PATCH, _WHALF), jnp.float32),
            pltpu.VMEM((8, _E), jnp.float32),
            pltpu.VMEM((8, _E), jnp.float32),
            pltpu.VMEM_SHARED((16, 8, _E), jnp.float32),
            pltpu.VMEM((8,), jnp.int32),
            pltpu.SemaphoreType.DMA,
            pltpu.SemaphoreType.DMA,
            pltpu.SemaphoreType.DMA,
            pltpu.SemaphoreType.DMA,
            pltpu.SemaphoreType.DMA,
            pltpu.SemaphoreType.DMA,
            pltpu.SemaphoreType.DMA,
        ],
    )(_sc_body)
    return run(x, table, idx)


# speedup vs baseline: 1.3114x; 1.1339x over previous
import functools

import jax
import jax.numpy as jnp
import numpy as np
from jax import lax
from jax.experimental import pallas as pl
from jax.experimental.pallas import tpu as pltpu
from jax.experimental.pallas import tpu_sc as plsc

_PATCH = 32
_GRID = 10
_H2 = 16
_W2 = 16
_E = _PATCH * _PATCH * 3
_WHALF = 256


def _hash_rows():
    i = np.arange(_H2)
    j = np.arange(_W2)
    hi = np.floor(i.astype(np.float32) * _GRID / _H2).astype(np.int32)
    hj = np.floor(j.astype(np.float32) * _GRID / _W2).astype(np.int32)
    return (hi[:, None] * _GRID + hj[None, :]).reshape(-1)


def _sc_body(x_hbm, tab_hbm, idx_hbm, out_hbm,
             in0, in1, out0, out1, shared, idx8_v,
             sem_i0, sem_i1, sem_o0, sem_o1, sem_p0, sem_p1, sem_t):
    cidx = lax.axis_index("c")
    sidx = lax.axis_index("s")
    wid = sidx * 2 + cidx
    ph = wid // 2
    half = wid % 2
    p0 = ph * _W2 + half * 8
    row0 = ph * _PATCH
    w0 = half * _WHALF
    nb = x_hbm.shape[0]

    def din(b, buf, sem):
        return pltpu.make_async_copy(
            x_hbm.at[b, :, pl.ds(row0, _PATCH), pl.ds(w0, _WHALF)], buf, sem)

    def dout(b, buf, sem):
        return pltpu.make_async_copy(buf, out_hbm.at[b, pl.ds(p0, 8)], sem)

    def prefill(buf, sem):
        return pltpu.make_async_copy(shared.at[sidx], buf, sem)

    din(0, in0, sem_i0).start()
    din(1, in1, sem_i1).start()

    pltpu.sync_copy(idx_hbm.at[pl.ds(p0, 8)], idx8_v)
    pltpu.async_copy(tab_hbm.at[idx8_v], out0, sem_t).wait()
    pltpu.sync_copy(out0, shared.at[sidx])

    prefill(out0, sem_p0).start()
    prefill(out1, sem_p1).start()

    iota3 = lax.iota(jnp.int32, 16) * 3
    rows = [jnp.full((16,), pw, jnp.int32) for pw in range(8)]

    def compute(in_v, out_v):
        for c in range(3):
            @plsc.parallel_loop(0, _PATCH, unroll=1)
            def _(kh, c=c):
                col0 = iota3 + (kh * 96 + c)
                col1 = col0 + 48
                for s in range(16):
                    row = rows[s // 2]
                    col = col0 if s % 2 == 0 else col1
                    v = in_v[c, kh, pl.ds(s * 16, 16)]
                    plsc.addupdate_scatter(out_v, [row, col], v)

    def phase(b, in_v, out_v, sem_i, sem_o, sem_p):
        din(0, in_v, sem_i).wait()
        prefill(out_v, sem_p).wait()
        compute(in_v, out_v)
        dout(b, out_v, sem_o).start()

        @pl.when(b + 2 < nb)
        def _():
            din(b + 2, in_v, sem_i).start()

    phase(0, in0, out0, sem_i0, sem_o0, sem_p0)
    phase(1, in1, out1, sem_i1, sem_o1, sem_p1)

    def steady(i, carry):
        dout(0, out0, sem_o0).wait()
        prefill(out0, sem_p0).start()
        dout(0, out1, sem_o1).wait()
        prefill(out1, sem_p1).start()
        phase(2 * i, in0, out0, sem_i0, sem_o0, sem_p0)
        phase(2 * i + 1, in1, out1, sem_i1, sem_o1, sem_p1)
        return carry

    lax.fori_loop(1, nb // 2, steady, 0)

    dout(0, out0, sem_o0).wait()
    dout(0, out1, sem_o1).wait()


def kernel(x, position_embeddings):
    b = x.shape[0]
    table = position_embeddings.reshape(_GRID * _GRID, _E)
    idx = jnp.asarray(_hash_rows())

    mesh = plsc.VectorSubcoreMesh(core_axis_name="c", subcore_axis_name="s")
    run = functools.partial(
        pl.kernel,
        out_type=jax.ShapeDtypeStruct((b, _H2 * _W2, _E), x.dtype),
        mesh=mesh,
        compiler_params=pltpu.CompilerParams(needs_layout_passes=False),
        scratch_types=[
            pltpu.VMEM((3, _PATCH, _WHALF), jnp.float32),
            pltpu.VMEM((3, _PATCH, _WHALF), jnp.float32),
            pltpu.VMEM((8, _E), jnp.float32),
            pltpu.VMEM((8, _E), jnp.float32),
            pltpu.VMEM_SHARED((16, 8, _E), jnp.float32),
            pltpu.VMEM((8,), jnp.int32),
            pltpu.SemaphoreType.DMA,
            pltpu.SemaphoreType.DMA,
            pltpu.SemaphoreType.DMA,
            pltpu.SemaphoreType.DMA,
            pltpu.SemaphoreType.DMA,
            pltpu.SemaphoreType.DMA,
            pltpu.SemaphoreType.DMA,
        ],
    )(_sc_body)
    return run(x, table, idx)
